# baseline (speedup 1.0000x reference)
import functools

import jax
import jax.numpy as jnp
from jax.experimental import pallas as pl
from jax.experimental.pallas import tpu as pltpu

EPS_DICE = 0.0001


def _dice_plane_kernel(x_ref, t_ref, o_ref, *, num_classes):
    i = pl.program_id(0)
    ci = jax.lax.rem(i, num_classes)
    x = x_ref[0]
    t = t_ref[0]
    h, w = x.shape
    m = t == ci

    def _rsum(v):
        part = jnp.sum(v.reshape(h // 8, 8, w), axis=0)
        return jnp.sum(part, keepdims=True).reshape(1, 1)

    t_tot = _rsum(jnp.tanh(x * 0.5))
    t_int = _rsum(jnp.where(m, -jnp.tanh(x * -0.5), 0.0))
    cnt = _rsum(jnp.where(m, 1.0, 0.0))
    o_ref[0] = jnp.concatenate([t_tot, t_int, cnt], axis=1)


@jax.jit
def kernel(output, target):
    b, c, h, w = output.shape
    x = output.reshape(b * c, h, w)
    tgt = target.astype(jnp.int32)
    acc = pl.pallas_call(
        functools.partial(_dice_plane_kernel, num_classes=c),
        grid=(b * c,),
        in_specs=[
            pl.BlockSpec((1, h, w), lambda i: (i, 0, 0)),
            pl.BlockSpec((1, h, w), lambda i: (i // c, 0, 0)),
        ],
        out_specs=pl.BlockSpec((1, 1, 3), lambda i: (i, 0, 0)),
        out_shape=jax.ShapeDtypeStruct((b * c, 1, 3), jnp.float32),
        compiler_params=pltpu.CompilerParams(
            dimension_semantics=("arbitrary",),
        ),
    )(x, tgt)
    acc = acc[:, 0, :].reshape(b, c, 3)
    t_tot = acc[:, :, 0]
    t_int = acc[:, :, 1]
    cnt = acc[:, :, 2]
    hw = jnp.float32(h * w)
    sig_sum = 0.5 * t_tot + 0.5 * hw
    inter = 0.5 * t_int + 0.5 * cnt
    loss_per_channel = 1.0 - 2.0 * inter / (sig_sum + cnt + EPS_DICE)
    return loss_per_channel.sum(axis=1) / c


# speedup vs baseline: 1.8817x; 1.8817x over previous
import jax
import jax.numpy as jnp
from jax.experimental import pallas as pl
from jax.experimental.pallas import tpu as pltpu

EPS_DICE = 0.0001


def _dice_block_kernel(out_ref, tgt_ref, acc_ref):
    x = out_ref[0]
    t = tgt_ref[0]
    c = x.shape[0]
    cls = jax.lax.broadcasted_iota(jnp.int32, x.shape, 0)
    t_tot = jnp.sum(jnp.tanh(x * 0.5), axis=(1, 2))
    s2 = jnp.sum(
        jnp.where(t[None, :, :] == cls, 1.0 - jnp.tanh(x * -0.5), 0.0),
        axis=(1, 2),
    )
    cnt = jnp.stack(
        [jnp.sum(jnp.where(t == ci, 1.0, 0.0)) for ci in range(c)]
    )
    acc_ref[0, 0] = jnp.concatenate([t_tot, s2, cnt])


@jax.jit
def kernel(output, target):
    b, c, h, w = output.shape
    tgt = target.astype(jnp.int32)
    acc = pl.pallas_call(
        _dice_block_kernel,
        grid=(b,),
        in_specs=[
            pl.BlockSpec((1, c, h, w), lambda i: (i, 0, 0, 0)),
            pl.BlockSpec((1, h, w), lambda i: (i, 0, 0)),
        ],
        out_specs=pl.BlockSpec((1, 1, 3 * c), lambda i: (i, 0, 0)),
        out_shape=jax.ShapeDtypeStruct((b, 1, 3 * c), jnp.float32),
        compiler_params=pltpu.CompilerParams(
            dimension_semantics=("arbitrary",),
        ),
    )(output, tgt)
    t_tot = acc[:, 0, :c]
    s2 = acc[:, 0, c : 2 * c]
    cnt = acc[:, 0, 2 * c :]
    hw = jnp.float32(h * w)
    sig_sum = 0.5 * t_tot + 0.5 * hw
    loss_per_channel = 1.0 - s2 / (sig_sum + cnt + EPS_DICE)
    return loss_per_channel.sum(axis=1) / c


# speedup vs baseline: 2.1288x; 1.1314x over previous
import jax
import jax.numpy as jnp
from jax.experimental import pallas as pl
from jax.experimental.pallas import tpu as pltpu

EPS_DICE = 0.0001


def _dice_block_kernel(out_ref, tgt_ref, acc_ref):
    x = out_ref[0]
    t = tgt_ref[0]
    cls = jax.lax.broadcasted_iota(jnp.int32, x.shape, 0)
    th = jnp.tanh(x * 0.5)
    m = t[None, :, :] == cls
    t_tot = jnp.sum(th, axis=(1, 2))
    t_int = jnp.sum(jnp.where(m, th, 0.0), axis=(1, 2))
    cnt = jnp.sum(jnp.where(m, 1.0, 0.0), axis=(1, 2))
    acc_ref[0, 0] = jnp.concatenate([t_tot, t_int, cnt])


@jax.jit
def kernel(output, target):
    b, c, h, w = output.shape
    tgt = target.astype(jnp.int32)
    acc = pl.pallas_call(
        _dice_block_kernel,
        grid=(b,),
        in_specs=[
            pl.BlockSpec((1, c, h, w), lambda i: (i, 0, 0, 0)),
            pl.BlockSpec((1, h, w), lambda i: (i, 0, 0)),
        ],
        out_specs=pl.BlockSpec((1, 1, 3 * c), lambda i: (i, 0, 0)),
        out_shape=jax.ShapeDtypeStruct((b, 1, 3 * c), jnp.float32),
        compiler_params=pltpu.CompilerParams(
            dimension_semantics=("arbitrary",),
            vmem_limit_bytes=100 * 1024 * 1024,
        ),
    )(output, tgt)
    t_tot = acc[:, 0, :c]
    t_int = acc[:, 0, c : 2 * c]
    cnt = acc[:, 0, 2 * c :]
    hw = jnp.float32(h * w)
    sig_sum = 0.5 * t_tot + 0.5 * hw
    inter = 0.5 * t_int + 0.5 * cnt
    loss_per_channel = 1.0 - 2.0 * inter / (sig_sum + cnt + EPS_DICE)
    return loss_per_channel.sum(axis=1) / c
